# baseline (speedup 1.0000x reference)
import functools
import math

import jax
import jax.numpy as jnp
from jax import lax
from jax.experimental import pallas as pl
from jax.experimental.pallas import tpu as pltpu
from jax.experimental.pallas import tpu_sc as plsc

_D = 64
_G = 128
_K = 5
_C = _G * _K
_SCALE = math.sqrt(_D)


def _make_gather(V, B):
    info = plsc.get_sparse_core_info()
    NC, NS = info.num_cores, info.num_subcores
    NW = NC * NS
    n_per_w = B // NW
    ng = n_per_w // _C
    assert n_per_w % _C == 0

    mesh = plsc.VectorSubcoreMesh(core_axis_name="c", subcore_axis_name="s")

    @functools.partial(
        pl.kernel,
        mesh=mesh,
        out_type=jax.ShapeDtypeStruct((B, _D), jnp.float32),
        scratch_types=[
            pltpu.VMEM((_C,), jnp.int32),
            pltpu.VMEM((_C, _D), jnp.float32),
            pltpu.SemaphoreType.DMA,
        ],
        compiler_params=pltpu.CompilerParams(use_tc_tiling_on_sc=False),
    )
    def gather_kernel(table_hbm, xidx_hbm, out_hbm, idx_v, rows_v, sem):
        wid = lax.axis_index("s") * NC + lax.axis_index("c")

        def chunk(g, carry):
            base = wid * n_per_w + g * _C
            pltpu.sync_copy(xidx_hbm.at[pl.ds(base, _C)], idx_v)
            cps = [
                pltpu.async_copy(
                    table_hbm.at[idx_v.at[pl.ds(j * _G, _G)]],
                    rows_v.at[pl.ds(j * _G, _G), :],
                    sem,
                )
                for j in range(_K)
            ]
            for cp in cps:
                cp.wait()

            def scale_row(r, c2):
                for c in range(_D // 16):
                    sl = pl.ds(c * 16, 16)
                    rows_v[r, sl] = rows_v[r, sl] * _SCALE
                return c2

            lax.fori_loop(0, _C, scale_row, 0)
            pltpu.sync_copy(rows_v, out_hbm.at[pl.ds(base, _C), :])
            return carry

        lax.fori_loop(0, ng, chunk, 0)

    return gather_kernel


def kernel(x, table):
    B_, H_ = x.shape
    V, D = table.shape
    B = B_ * H_
    xf = x.reshape(-1).astype(jnp.int32)
    out = _make_gather(V, B)(table, xf)
    return out.reshape(B_, H_, D)
